# Initial kernel scaffold; baseline (speedup 1.0000x reference)
import jax
import jax.numpy as jnp
from jax.experimental import pallas as pl


def kernel(coords, demands, capacity, W_amp, b_amp, W1, b1, W2, b2):
    raise NotImplementedError("write your pallas kernel here")


# speedup vs baseline: 12.5036x; 12.5036x over previous
import functools

import jax
import jax.numpy as jnp
from jax import lax
from jax.experimental import pallas as pl
from jax.experimental.pallas import tpu as pltpu

_B, _N, _K, _HID = 64, 1024, 10, 16


def _tc_body(cap_ref, wamp_ref, bamp_ref, w1_ref, b1_ref, w2_ref, b2_ref,
             xs_ref, ys_ref, dm_ref, co_ref,
             psix_ref, psiy_ref, nd_ref, dd_ref, knn_ref):
    x = xs_ref[0]
    y = ys_ref[0]
    dm = dm_ref[0]
    cap = cap_ref[0, 0]

    x0 = xs_ref[0, 0:1, 0:1]
    y0 = ys_ref[0, 0:1, 0:1]
    dx0 = x - x0
    dy0 = y - y0
    dd = jnp.sqrt(dx0 * dx0 + dy0 * dy0 + jnp.float32(1e-12))
    nd = dm / cap
    lane = lax.broadcasted_iota(jnp.int32, (1, _N), 1)
    isd = jnp.where(lane == 0, jnp.float32(1.0), jnp.float32(0.0))
    feats = (x, y, nd, dd, isd)

    nd_ref[0] = nd
    dd_ref[0] = dd

    p0 = bamp_ref[0, 0]
    p1 = bamp_ref[0, 1]
    for d in range(5):
        p0 = p0 + feats[d] * wamp_ref[d, 0]
        p1 = p1 + feats[d] * wamp_ref[d, 1]
    norm = jnp.sqrt(p0 * p0 + p1 * p1) + jnp.float32(1e-8)
    p0 = p0 / norm
    p1 = p1 / norm

    theta = b2_ref[0, 0]
    for j in range(_HID):
        hj = b1_ref[0, j]
        for d in range(5):
            hj = hj + feats[d] * w1_ref[d, j]
        theta = theta + jnp.tanh(hj) * w2_ref[j, 0]
    c = jnp.cos(theta)
    s = jnp.sin(theta)
    psix_ref[0] = c * p0 - s * p1
    psiy_ref[0] = s * p0 + c * p1

    xc = co_ref[0, :, 0:1]
    yc = co_ref[0, :, 1:2]
    dx = xc - x
    dy = yc - y
    d2 = dx * dx + dy * dy
    ri = lax.broadcasted_iota(jnp.int32, (_N, _N), 0)
    ci = lax.broadcasted_iota(jnp.int32, (_N, _N), 1)
    d2 = jnp.where(ri == ci, jnp.float32(1e9), d2)
    for r in range(_K):
        m = jnp.min(d2, axis=0, keepdims=True)
        am = jnp.min(jnp.where(d2 == m, ri, _N), axis=0, keepdims=True)
        knn_ref[0, r, :] = am[0, :]
        d2 = jnp.where(ri == am, jnp.float32(1e9), d2)


@jax.jit
def _run(coords, xs, ys, demands, capacity, W_amp, b_amp, W1, b1, W2, b2):
    smem = pl.BlockSpec(memory_space=pltpu.SMEM)
    row = pl.BlockSpec((1, 1, _N), lambda b: (b, 0, 0))
    psix, psiy, nd, dd, knn_t = pl.pallas_call(
        _tc_body,
        grid=(_B,),
        in_specs=[smem] * 7 + [row, row, row,
                               pl.BlockSpec((1, _N, 2), lambda b: (b, 0, 0))],
        out_specs=[row, row, row, row,
                   pl.BlockSpec((1, _K, _N), lambda b: (b, 0, 0))],
        out_shape=[
            jax.ShapeDtypeStruct((_B, 1, _N), jnp.float32),
            jax.ShapeDtypeStruct((_B, 1, _N), jnp.float32),
            jax.ShapeDtypeStruct((_B, 1, _N), jnp.float32),
            jax.ShapeDtypeStruct((_B, 1, _N), jnp.float32),
            jax.ShapeDtypeStruct((_B, _K, _N), jnp.int32),
        ],
    )(capacity.reshape(1, 1), W_amp, b_amp.reshape(1, 2), W1,
      b1.reshape(1, _HID), W2, b2.reshape(1, 1),
      xs.reshape(_B, 1, _N), ys.reshape(_B, 1, _N),
      demands.reshape(_B, 1, _N), coords)
    return (psix.reshape(_B, _N), psiy.reshape(_B, _N),
            nd.reshape(_B, _N), dd.reshape(_B, _N), knn_t)


def kernel(coords, demands, capacity, W_amp, b_amp, W1, b1, W2, b2):
    xs = coords[:, :, 0]
    ys = coords[:, :, 1]
    psix, psiy, nd, dd, knn_t = _run(
        coords, xs, ys, demands, capacity, W_amp, b_amp, W1, b1, W2, b2)
    isd = jnp.zeros((_B, _N), jnp.float32).at[:, 0].set(1.0)
    features = jnp.stack([xs, ys, nd, dd, isd], axis=-1)
    psi_prime = jnp.stack([psix, psiy], axis=-1)
    knn = knn_t.transpose(0, 2, 1)
    return psi_prime, features, knn
